# baseline (speedup 1.0000x reference)
import jax
import jax.numpy as jnp
from jax.experimental import pallas as pl


def _select_kernel(x_ref, m_ref, o_ref):
    o_ref[...] = jnp.where(m_ref[...], x_ref[...], 0.0)


def kernel(x, mask):
    B, N = x.shape
    BLK = 4096
    grid = (N // BLK,)
    return pl.pallas_call(
        _select_kernel,
        grid=grid,
        in_specs=[
            pl.BlockSpec((B, BLK), lambda j: (0, j)),
            pl.BlockSpec((B, BLK), lambda j: (0, j)),
        ],
        out_specs=pl.BlockSpec((B, BLK), lambda j: (0, j)),
        out_shape=jax.ShapeDtypeStruct((B, N), x.dtype),
    )(x, mask)
